# baseline (speedup 1.0000x reference)
import jax
import jax.numpy as jnp
from jax import lax
from jax.experimental import pallas as pl
from jax.experimental.pallas import tpu as pltpu
from jax.experimental.pallas import tpu_sc as plsc

SEQ_LEN = 200
BATCH = 4096
VOCAB = 1000000
EMBED_DIM = 64
OUTPUT_DIM = 2
PAD_O = 8
NUM_CORES = 2
NUM_SUBCORES = 16
NUM_WORKERS = NUM_CORES * NUM_SUBCORES
B_PER_W = BATCH // NUM_WORKERS
PROJ_BLK = 8192


def _proj_body(w_ref, x_ref, b_ref, o_ref):
  o_ref[...] = (
      jnp.dot(w_ref[...], x_ref[...], preferred_element_type=jnp.float32)
      + b_ref[...]
  )


def _project_table(table_t, w_scaled, b_scaled):
  grid = (VOCAB + PROJ_BLK - 1) // PROJ_BLK
  return pl.pallas_call(
      _proj_body,
      grid=(grid,),
      in_specs=[
          pl.BlockSpec((PAD_O, EMBED_DIM), lambda i: (0, 0)),
          pl.BlockSpec((EMBED_DIM, PROJ_BLK), lambda i: (0, i)),
          pl.BlockSpec((PAD_O, 1), lambda i: (0, 0)),
      ],
      out_specs=pl.BlockSpec((PAD_O, PROJ_BLK), lambda i: (0, i)),
      out_shape=jax.ShapeDtypeStruct((PAD_O, VOCAB), jnp.float32),
  )(w_scaled, table_t, b_scaled)


def _sc_body(text_ref, proj_ref, slots_ref, out_ref,
             idx_v, rows0, rows1, acc, sidx, sem0, sem1):
  sid = lax.axis_index("s")
  wid = sid * NUM_CORES + lax.axis_index("c")
  base = wid * B_PER_W

  pltpu.sync_copy(text_ref.at[:, pl.ds(base, B_PER_W)], idx_v)
  pltpu.sync_copy(slots_ref.at[sid], sidx)

  def start(t, buf, sem):
    pltpu.async_copy(proj_ref.at[idx_v.at[t]], buf.at[0], sem)

  def wait(buf, sem):
    pltpu.make_async_copy(proj_ref.at[idx_v.at[0]], buf.at[0], sem).wait()

  def accum(buf, first=False):
    if first:
      pltpu.sync_copy(buf, acc.at[sidx])
    else:
      pltpu.sync_copy(buf, acc.at[sidx], add=True)

  start(0, rows0, sem0)
  start(1, rows1, sem1)
  wait(rows0, sem0)
  accum(rows0, first=True)
  start(2, rows0, sem0)
  wait(rows1, sem1)
  accum(rows1)
  start(3, rows1, sem1)

  @pl.loop(1, SEQ_LEN // 2 - 1)
  def _(g):
    wait(rows0, sem0)
    accum(rows0)
    start(2 * g + 2, rows0, sem0)
    wait(rows1, sem1)
    accum(rows1)
    start(2 * g + 3, rows1, sem1)

  wait(rows0, sem0)
  accum(rows0)
  wait(rows1, sem1)
  accum(rows1)

  pltpu.sync_copy(acc.at[sid], out_ref.at[pl.ds(base, B_PER_W)])


def _sc_embed_bag(text, proj, slot_ids):
  mesh = plsc.VectorSubcoreMesh(core_axis_name="c", subcore_axis_name="s")
  return pl.kernel(
      _sc_body,
      out_type=jax.ShapeDtypeStruct((BATCH, PAD_O), jnp.float32),
      mesh=mesh,
      scratch_types=[
          pltpu.VMEM((SEQ_LEN, B_PER_W), jnp.int32),
          pltpu.VMEM((1, B_PER_W, PAD_O), jnp.float32),
          pltpu.VMEM((1, B_PER_W, PAD_O), jnp.float32),
          pltpu.VMEM_SHARED((NUM_SUBCORES, B_PER_W, PAD_O), jnp.float32),
          pltpu.VMEM((1,), jnp.int32),
          pltpu.SemaphoreType.DMA,
          pltpu.SemaphoreType.DMA,
      ],
      compiler_params=pltpu.CompilerParams(use_tc_tiling_on_sc=False),
  )(text, proj, slot_ids)


@jax.jit
def kernel(text, embed_table, fc_w, fc_b):
  text = text.astype(jnp.int32)
  inv = jnp.float32(1.0 / SEQ_LEN)
  w_scaled = jnp.zeros((PAD_O, EMBED_DIM), jnp.float32).at[:OUTPUT_DIM].set(
      fc_w * inv)
  b_scaled = jnp.zeros((PAD_O, 1), jnp.float32).at[:OUTPUT_DIM, 0].set(
      fc_b * inv)
  proj_t = _project_table(embed_table.T, w_scaled, b_scaled)
  proj = proj_t.T
  slot_ids = jnp.arange(NUM_SUBCORES, dtype=jnp.int32).reshape(NUM_SUBCORES, 1)
  out = _sc_embed_bag(text, proj, slot_ids)
  return out[:, :OUTPUT_DIM]


# speedup vs baseline: 3.1584x; 3.1584x over previous
import jax
import jax.numpy as jnp
from jax import lax
from jax.experimental import pallas as pl
from jax.experimental.pallas import tpu as pltpu
from jax.experimental.pallas import tpu_sc as plsc

SEQ_LEN = 200
BATCH = 4096
VOCAB = 1000000
EMBED_DIM = 64
OUTPUT_DIM = 2
NUM_CORES = 2
NUM_SUBCORES = 16
NUM_WORKERS = NUM_CORES * NUM_SUBCORES
B_PER_W = BATCH // NUM_WORKERS
PROJ_BLK = 8192


def _proj_body(w_ref, x_ref, b_ref, o0_ref, o1_ref):
  y = (
      jnp.dot(w_ref[...], x_ref[...], preferred_element_type=jnp.float32)
      + b_ref[...]
  )
  o0_ref[...] = y[0]
  o1_ref[...] = y[1]


def _project_table(table_t, w_scaled, b_scaled):
  grid = (VOCAB + PROJ_BLK - 1) // PROJ_BLK
  vec = jax.ShapeDtypeStruct((VOCAB,), jnp.float32)
  return pl.pallas_call(
      _proj_body,
      grid=(grid,),
      in_specs=[
          pl.BlockSpec((OUTPUT_DIM, EMBED_DIM), lambda i: (0, 0)),
          pl.BlockSpec((EMBED_DIM, PROJ_BLK), lambda i: (0, i)),
          pl.BlockSpec((OUTPUT_DIM, 1), lambda i: (0, 0)),
      ],
      out_specs=[
          pl.BlockSpec((PROJ_BLK,), lambda i: (i,)),
          pl.BlockSpec((PROJ_BLK,), lambda i: (i,)),
      ],
      out_shape=[vec, vec],
  )(w_scaled, table_t, b_scaled)


def _sc_body(text_ref, p0_ref, p1_ref, slots_ref, out_ref,
             idx_v, rows0, rows1, acc, sidx, sem0, sem1):
  sid = lax.axis_index("s")
  wid = sid * NUM_CORES + lax.axis_index("c")
  base = wid * B_PER_W

  pltpu.sync_copy(text_ref.at[:, pl.ds(base, B_PER_W)], idx_v)
  pltpu.sync_copy(slots_ref.at[sid], sidx)

  def start(t, buf, sem):
    pltpu.async_copy(p0_ref.at[idx_v.at[t]], buf.at[0, 0], sem)
    pltpu.async_copy(p1_ref.at[idx_v.at[t]], buf.at[0, 1], sem)

  def wait(buf, sem):
    pltpu.make_async_copy(p0_ref.at[idx_v.at[0]], buf.at[0, 0], sem).wait()
    pltpu.make_async_copy(p1_ref.at[idx_v.at[0]], buf.at[0, 1], sem).wait()

  def accum(buf, first=False):
    if first:
      pltpu.sync_copy(buf, acc.at[sidx])
    else:
      pltpu.sync_copy(buf, acc.at[sidx], add=True)

  start(0, rows0, sem0)
  start(1, rows1, sem1)
  wait(rows0, sem0)
  accum(rows0, first=True)
  start(2, rows0, sem0)
  wait(rows1, sem1)
  accum(rows1)
  start(3, rows1, sem1)

  @pl.loop(1, SEQ_LEN // 2 - 1)
  def _(g):
    wait(rows0, sem0)
    accum(rows0)
    start(2 * g + 2, rows0, sem0)
    wait(rows1, sem1)
    accum(rows1)
    start(2 * g + 3, rows1, sem1)

  wait(rows0, sem0)
  accum(rows0)
  wait(rows1, sem1)
  accum(rows1)

  pltpu.sync_copy(acc.at[sid], out_ref.at[:, pl.ds(base, B_PER_W)])


def _sc_embed_bag(text, proj0, proj1, slot_ids):
  mesh = plsc.VectorSubcoreMesh(core_axis_name="c", subcore_axis_name="s")
  return pl.kernel(
      _sc_body,
      out_type=jax.ShapeDtypeStruct((OUTPUT_DIM, BATCH), jnp.float32),
      mesh=mesh,
      scratch_types=[
          pltpu.VMEM((SEQ_LEN, B_PER_W), jnp.int32),
          pltpu.VMEM((1, OUTPUT_DIM, B_PER_W), jnp.float32),
          pltpu.VMEM((1, OUTPUT_DIM, B_PER_W), jnp.float32),
          pltpu.VMEM_SHARED((NUM_SUBCORES, OUTPUT_DIM, B_PER_W), jnp.float32),
          pltpu.VMEM((1,), jnp.int32),
          pltpu.SemaphoreType.DMA,
          pltpu.SemaphoreType.DMA,
      ],
      compiler_params=pltpu.CompilerParams(use_tc_tiling_on_sc=False),
  )(text, proj0, proj1, slot_ids)


@jax.jit
def kernel(text, embed_table, fc_w, fc_b):
  text = text.astype(jnp.int32)
  inv = jnp.float32(1.0 / SEQ_LEN)
  w_scaled = fc_w * inv
  b_scaled = (fc_b * inv).reshape(OUTPUT_DIM, 1)
  proj0, proj1 = _project_table(embed_table.T, w_scaled, b_scaled)
  slot_ids = jnp.arange(NUM_SUBCORES, dtype=jnp.int32).reshape(NUM_SUBCORES, 1)
  out = _sc_embed_bag(text, proj0, proj1, slot_ids)
  return out.T


# speedup vs baseline: 3.8467x; 1.2179x over previous
import jax
import jax.numpy as jnp
from jax import lax
from jax.experimental import pallas as pl
from jax.experimental.pallas import tpu as pltpu
from jax.experimental.pallas import tpu_sc as plsc

SEQ_LEN = 200
BATCH = 4096
VOCAB = 1000000
EMBED_DIM = 64
OUTPUT_DIM = 2
NUM_CORES = 2
NUM_SUBCORES = 16
NUM_WORKERS = NUM_CORES * NUM_SUBCORES
B_PER_W = BATCH // NUM_WORKERS
PROJ_BLK = 32768


def _proj_body(w_ref, x_ref, b_ref, o0_ref, o1_ref):
  y = (
      jnp.dot(w_ref[...], x_ref[...], preferred_element_type=jnp.float32)
      + b_ref[...]
  )
  o0_ref[...] = y[0]
  o1_ref[...] = y[1]


def _project_table(table_t, w_scaled, b_scaled):
  grid = (VOCAB + PROJ_BLK - 1) // PROJ_BLK
  vec = jax.ShapeDtypeStruct((VOCAB,), jnp.float32)
  return pl.pallas_call(
      _proj_body,
      grid=(grid,),
      in_specs=[
          pl.BlockSpec((OUTPUT_DIM, EMBED_DIM), lambda i: (0, 0)),
          pl.BlockSpec((EMBED_DIM, PROJ_BLK), lambda i: (0, i)),
          pl.BlockSpec((OUTPUT_DIM, 1), lambda i: (0, 0)),
      ],
      out_specs=[
          pl.BlockSpec((PROJ_BLK,), lambda i: (i,)),
          pl.BlockSpec((PROJ_BLK,), lambda i: (i,)),
      ],
      out_shape=[vec, vec],
  )(w_scaled, table_t, b_scaled)


def _sc_body(text_ref, p0_ref, p1_ref, slots_ref, out_ref,
             idx_v, rows0, rows1, acc, sidx, sem0, sem1):
  sid = lax.axis_index("s")
  wid = sid * NUM_CORES + lax.axis_index("c")
  base = wid * B_PER_W

  pltpu.sync_copy(text_ref.at[:, pl.ds(base, B_PER_W)], idx_v)
  pltpu.sync_copy(slots_ref.at[sid], sidx)

  def start(t, buf, sem):
    pltpu.async_copy(p0_ref.at[idx_v.at[t]], buf.at[0, 0], sem)
    pltpu.async_copy(p1_ref.at[idx_v.at[t]], buf.at[0, 1], sem)

  def wait(buf, sem):
    pltpu.make_async_copy(p0_ref.at[idx_v.at[0]], buf.at[0, 0], sem).wait()
    pltpu.make_async_copy(p1_ref.at[idx_v.at[0]], buf.at[0, 1], sem).wait()

  def accum(buf, first=False):
    if first:
      pltpu.sync_copy(buf, acc.at[sidx])
    else:
      pltpu.sync_copy(buf, acc.at[sidx], add=True)

  start(0, rows0, sem0)
  start(1, rows1, sem1)
  wait(rows0, sem0)
  accum(rows0, first=True)
  start(2, rows0, sem0)
  wait(rows1, sem1)
  accum(rows1)
  start(3, rows1, sem1)

  @pl.loop(1, SEQ_LEN // 2 - 1)
  def _(g):
    wait(rows0, sem0)
    accum(rows0)
    start(2 * g + 2, rows0, sem0)
    wait(rows1, sem1)
    accum(rows1)
    start(2 * g + 3, rows1, sem1)

  wait(rows0, sem0)
  accum(rows0)
  wait(rows1, sem1)
  accum(rows1)

  pltpu.sync_copy(acc.at[sid], out_ref.at[:, pl.ds(base, B_PER_W)])


def _sc_embed_bag(text, proj0, proj1, slot_ids):
  mesh = plsc.VectorSubcoreMesh(core_axis_name="c", subcore_axis_name="s")
  return pl.kernel(
      _sc_body,
      out_type=jax.ShapeDtypeStruct((OUTPUT_DIM, BATCH), jnp.float32),
      mesh=mesh,
      scratch_types=[
          pltpu.VMEM((SEQ_LEN, B_PER_W), jnp.int32),
          pltpu.VMEM((1, OUTPUT_DIM, B_PER_W), jnp.float32),
          pltpu.VMEM((1, OUTPUT_DIM, B_PER_W), jnp.float32),
          pltpu.VMEM_SHARED((NUM_SUBCORES, OUTPUT_DIM, B_PER_W), jnp.float32),
          pltpu.VMEM((1,), jnp.int32),
          pltpu.SemaphoreType.DMA,
          pltpu.SemaphoreType.DMA,
      ],
      compiler_params=pltpu.CompilerParams(use_tc_tiling_on_sc=False),
  )(text, proj0, proj1, slot_ids)


@jax.jit
def kernel(text, embed_table, fc_w, fc_b):
  text = text.astype(jnp.int32)
  inv = jnp.float32(1.0 / SEQ_LEN)
  w_scaled = fc_w * inv
  b_scaled = (fc_b * inv).reshape(OUTPUT_DIM, 1)
  proj0, proj1 = _project_table(embed_table.T, w_scaled, b_scaled)
  slot_ids = jnp.arange(NUM_SUBCORES, dtype=jnp.int32).reshape(NUM_SUBCORES, 1)
  out = _sc_embed_bag(text, proj0, proj1, slot_ids)
  return out.T
